# baseline (device time: 75231 ns/iter reference)
import os

import jax
import jax.numpy as jnp
from jax import lax
from jax.experimental import pallas as pl
from jax.experimental.pallas import tpu as pltpu

CHUNKS = (256,) * 8
VARIANT = os.environ.get("KVARIANT", "full")


def kernel(x, dy):
    m, d = x.shape
    _, f = dy.shape
    half_d = d // 2
    half_f = f // 2
    n_chunk = len(CHUNKS)
    offs = [sum(CHUNKS[:i]) for i in range(n_chunk)]
    cmax = max(CHUNKS)
    assert sum(CHUNKS) == half_f

    def body(x_ref, dy_ref, out_ref, xt_ref, psend_ref, plocal_ref,
             yrecv_ref, xsend_ref, xrecv_ref,
             y_send_sems, y_recv_sems, x_send_sems, x_recv_sems):
        my_x = lax.axis_index("x")
        my_y = lax.axis_index("y")
        col0 = my_x * half_f
        ocol0 = (1 - my_x) * half_f

        barrier_sem = pltpu.get_barrier_semaphore()
        pl.semaphore_signal(barrier_sem, inc=1,
                            device_id=(my_x, 1 - my_y),
                            device_id_type=pl.DeviceIdType.MESH)
        pl.semaphore_signal(barrier_sem, inc=1,
                            device_id=(1 - my_x, my_y),
                            device_id_type=pl.DeviceIdType.MESH)
        pl.semaphore_wait(barrier_sem, 2)

        other_rows = pl.ds((1 - my_y) * half_d, half_d)
        my_rows = pl.ds(my_y * half_d, half_d)

        if VARIANT != "nocompute":
            xt_ref[other_rows, :] = x_ref[:, other_rows].T
        y_rdmas = []
        for c in range(n_chunk):
            cf, off = CHUNKS[c], offs[c]
            if VARIANT == "nocompute":
                psend_ref[c, :, :] = dy_ref[:half_d, pl.ds(col0 + off, cf)]
            else:
                psend_ref[c, :, :] = lax.dot_general(
                    xt_ref[other_rows, :],
                    dy_ref[:, pl.ds(col0 + off, cf)],
                    (((1,), (0,)), ((), ())),
                    preferred_element_type=jnp.float32,
                )
            if VARIANT == "nocomm":
                continue
            rdma_y = pltpu.make_async_remote_copy(
                src_ref=psend_ref.at[c, :, :],
                dst_ref=yrecv_ref.at[c, :, :],
                send_sem=y_send_sems.at[c],
                recv_sem=y_recv_sems.at[c],
                device_id=(my_x, 1 - my_y),
                device_id_type=pl.DeviceIdType.MESH,
            )
            rdma_y.start()
            y_rdmas.append(rdma_y)

        if VARIANT != "nocompute":
            xt_ref[my_rows, :] = x_ref[:, my_rows].T
        for c in range(n_chunk):
            cf, off = CHUNKS[c], offs[c]
            if VARIANT == "nocompute":
                plocal_ref[c, :, :] = dy_ref[half_d:, pl.ds(col0 + off, cf)]
            else:
                plocal_ref[c, :, :] = lax.dot_general(
                    xt_ref[my_rows, :],
                    dy_ref[:, pl.ds(col0 + off, cf)],
                    (((1,), (0,)), ((), ())),
                    preferred_element_type=jnp.float32,
                )

        x_rdmas = []
        for c in range(n_chunk):
            cf, off = CHUNKS[c], offs[c]
            if VARIANT != "nocomm":
                y_rdmas[c].wait_recv()
                xsend_ref[c, :, :] = (
                    plocal_ref[c, :, :] + yrecv_ref[c, :, :]
                )
            else:
                xsend_ref[c, :, :] = (
                    plocal_ref[c, :, :] + psend_ref[c, :, :]
                )
            out_ref[:, pl.ds(col0 + off, cf)] = xsend_ref[c, :, :]
            if VARIANT in ("nocomm", "noxcomm"):
                out_ref[:, pl.ds(ocol0 + off, cf)] = xsend_ref[c, :, :]
                continue
            rdma_x = pltpu.make_async_remote_copy(
                src_ref=xsend_ref.at[c, :, :],
                dst_ref=xrecv_ref.at[c, :, :],
                send_sem=x_send_sems.at[c],
                recv_sem=x_recv_sems.at[c],
                device_id=(1 - my_x, my_y),
                device_id_type=pl.DeviceIdType.MESH,
            )
            rdma_x.start()
            x_rdmas.append(rdma_x)

        for c, rdma_x in enumerate(x_rdmas):
            cf, off = CHUNKS[c], offs[c]
            rdma_x.wait_recv()
            out_ref[:, pl.ds(ocol0 + off, cf)] = xrecv_ref[c, :, :]

        for rdma_y in y_rdmas:
            rdma_y.wait_send()
        for rdma_x in x_rdmas:
            rdma_x.wait_send()

    return pl.pallas_call(
        body,
        out_shape=jax.ShapeDtypeStruct((half_d, f), jnp.float32),
        in_specs=[
            pl.BlockSpec(memory_space=pltpu.VMEM),
            pl.BlockSpec(memory_space=pltpu.VMEM),
        ],
        out_specs=pl.BlockSpec(memory_space=pltpu.VMEM),
        scratch_shapes=[
            pltpu.VMEM((d, m), jnp.float32),
            pltpu.VMEM((n_chunk, half_d, cmax), jnp.float32),
            pltpu.VMEM((n_chunk, half_d, cmax), jnp.float32),
            pltpu.VMEM((n_chunk, half_d, cmax), jnp.float32),
            pltpu.VMEM((n_chunk, half_d, cmax), jnp.float32),
            pltpu.VMEM((n_chunk, half_d, cmax), jnp.float32),
            pltpu.SemaphoreType.DMA((n_chunk,)),
            pltpu.SemaphoreType.DMA((n_chunk,)),
            pltpu.SemaphoreType.DMA((n_chunk,)),
            pltpu.SemaphoreType.DMA((n_chunk,)),
        ],
        compiler_params=pltpu.CompilerParams(
            collective_id=0, vmem_limit_bytes=100 * 1024 * 1024
        ),
    )(x, dy)
